# baseline (speedup 1.0000x reference)
import jax
import jax.numpy as jnp
from jax.experimental import pallas as pl

_BETA = 0.25
_K = 1024
_D = 64
_T = 1024


def _vq_body(x_ref, e_ref, q_ref, loss_ref):
    e = e_ref[:]
    xb = x_ref[:]
    sim = jnp.dot(xb, e, preferred_element_type=jnp.float32)
    x2 = jnp.sum(xb * xb, axis=1, keepdims=True)
    e2 = jnp.sum(e * e, axis=0, keepdims=True)
    dist = (x2 + e2) - 2.0 * sim
    iota = jax.lax.broadcasted_iota(jnp.int32, (_T, _K), 1)
    m = jnp.min(dist, axis=1, keepdims=True)
    idx = jnp.min(jnp.where(dist == m, iota, _K), axis=1)
    onehot = (iota == idx[:, None]).astype(jnp.float32)
    q = jax.lax.dot_general(
        onehot, e, (((1,), (1,)), ((), ())),
        preferred_element_type=jnp.float32,
        precision=jax.lax.Precision.HIGHEST,
    )
    q_ref[:] = xb + (q - xb)

    part = jnp.sum((q - xb) ** 2)

    @pl.when(pl.program_id(0) == 0)
    def _():
        loss_ref[:, :] = jnp.zeros((1, 1), jnp.float32)

    loss_ref[:, :] += jnp.full((1, 1), part)


def kernel(x, embeddings):
    n = x.shape[0] * x.shape[1] * x.shape[2]
    xf = x.reshape(n, _D)
    q, loss_sum = pl.pallas_call(
        _vq_body,
        grid=(n // _T,),
        in_specs=[
            pl.BlockSpec((_T, _D), lambda i: (i, 0)),
            pl.BlockSpec((_D, _K), lambda i: (0, 0)),
        ],
        out_specs=[
            pl.BlockSpec((_T, _D), lambda i: (i, 0)),
            pl.BlockSpec((1, 1), lambda i: (0, 0)),
        ],
        out_shape=[
            jax.ShapeDtypeStruct((n, _D), jnp.float32),
            jax.ShapeDtypeStruct((1, 1), jnp.float32),
        ],
    )(xf, embeddings)
    vq_loss = loss_sum[0, 0] * ((1.0 + _BETA) / (n * _D))
    return q.reshape(x.shape), vq_loss


# speedup vs baseline: 1.1844x; 1.1844x over previous
import functools

import jax
import jax.numpy as jnp
from jax import lax
from jax.experimental import pallas as pl
from jax.experimental.pallas import tpu as pltpu
from jax.experimental.pallas import tpu_sc as plsc

_BETA = 0.25
_K = 1024
_D = 64
_T = 1024
_N = 16384

_NC = 2
_NS = 16
_NW = _NC * _NS
_BW = _N // _NW
_CHUNK = 128
_NCH = _BW // _CHUNK


def _vq_tc_body(x_ref, e_ref, idx_ref, loss_ref, et_ref):
    e = e_ref[:]
    xb = x_ref[:]
    sim = jnp.dot(xb, e, preferred_element_type=jnp.float32)
    x2 = jnp.sum(xb * xb, axis=1, keepdims=True)
    e2 = jnp.sum(e * e, axis=0, keepdims=True)
    dist = (x2 + e2) - 2.0 * sim
    iota = jax.lax.broadcasted_iota(jnp.int32, (_T, _K), 1)
    m = jnp.min(dist, axis=1, keepdims=True)
    idx = jnp.min(jnp.where(dist == m, iota, _K), axis=1)
    idx_ref[:, :, :] = idx[None, None, :]

    part = jnp.sum(m)

    @pl.when(pl.program_id(0) == 0)
    def _():
        loss_ref[:, :] = jnp.zeros((1, 1), jnp.float32)
        et_ref[:, :] = jnp.concatenate(
            [e.T, jnp.zeros((_K, 128 - _D), jnp.float32)], axis=1
        )

    loss_ref[:, :] += jnp.full((1, 1), part)


def _tc_stage(xf, embeddings):
    return pl.pallas_call(
        _vq_tc_body,
        grid=(_N // _T,),
        in_specs=[
            pl.BlockSpec((_T, _D), lambda i: (i, 0)),
            pl.BlockSpec((_D, _K), lambda i: (0, 0)),
        ],
        out_specs=[
            pl.BlockSpec((1, 1, _T), lambda i: (i, 0, 0)),
            pl.BlockSpec((1, 1), lambda i: (0, 0)),
            pl.BlockSpec((_K, 128), lambda i: (0, 0)),
        ],
        out_shape=[
            jax.ShapeDtypeStruct((_N // _T, 1, _T), jnp.int32),
            jax.ShapeDtypeStruct((1, 1), jnp.float32),
            jax.ShapeDtypeStruct((_K, 128), jnp.float32),
        ],
    )(xf, embeddings)


_sc_mesh = plsc.VectorSubcoreMesh(core_axis_name="c", subcore_axis_name="s")


@functools.partial(
    pl.kernel,
    out_type=jax.ShapeDtypeStruct((_N, _D), jnp.float32),
    mesh=_sc_mesh,
    scratch_types=[
        pltpu.VMEM((_NCH, _CHUNK), jnp.int32),
        pltpu.VMEM((_CHUNK, 128), jnp.float32),
        pltpu.VMEM((_BW, _D), jnp.float32),
        pltpu.SemaphoreType.DMA,
    ],
)
def _sc_gather(et_hbm, idx_hbm, out_hbm, idx_v, rows_v, out_v, sem):
    wid = lax.axis_index("s") * _NC + lax.axis_index("c")
    base = wid * _BW
    pltpu.sync_copy(idx_hbm.at[pl.ds(wid * _NCH, _NCH)], idx_v)
    for j in range(_NCH):
        pltpu.async_copy(et_hbm.at[idx_v.at[j]], rows_v, sem).wait()

        def _compact(t, _, j=j):
            for k in range(_D // 16):
                out_v[j * _CHUNK + t, pl.ds(k * 16, 16)] = rows_v[
                    t, pl.ds(k * 16, 16)
                ]
            return 0

        lax.fori_loop(0, _CHUNK, _compact, 0)
    pltpu.sync_copy(out_v, out_hbm.at[pl.ds(base, _BW)])


def kernel(x, embeddings):
    xf = x.reshape(_N, _D)
    idx, loss_sum, et = _tc_stage(xf, embeddings)
    q = _sc_gather(et, idx.reshape(_N // _CHUNK, _CHUNK))
    vq_loss = loss_sum[0, 0] * ((1.0 + _BETA) / (_N * _D))
    return q.reshape(x.shape), vq_loss


# speedup vs baseline: 1.2542x; 1.0590x over previous
import functools

import jax
import jax.numpy as jnp
from jax import lax
from jax.experimental import pallas as pl
from jax.experimental.pallas import tpu as pltpu
from jax.experimental.pallas import tpu_sc as plsc

_BETA = 0.25
_K = 1024
_D = 64
_T = 2048
_N = 16384

_NC = 2
_NS = 16
_NW = _NC * _NS
_BW = _N // _NW
_CHUNK = 128
_NCH = _BW // _CHUNK


def _vq_tc_body(x_ref, e_ref, idx_ref, loss_ref):
    e = e_ref[:]
    xb = x_ref[:]
    sim2 = jnp.dot(xb * -2.0, e, preferred_element_type=jnp.float32)
    x2 = jnp.sum(xb * xb, axis=1, keepdims=True)
    e2 = jnp.sum(e * e, axis=0, keepdims=True)
    dist = (x2 + e2) + sim2
    iota = jax.lax.broadcasted_iota(jnp.int32, (_T, _K), 1)
    m = jnp.min(dist, axis=1, keepdims=True)
    idx = jnp.min(jnp.where(dist == m, iota, _K), axis=1)
    idx_ref[:, :, :] = idx[None, None, :]

    part = jnp.sum(m)

    @pl.when(pl.program_id(0) == 0)
    def _():
        loss_ref[:, :] = jnp.zeros((1, 1), jnp.float32)

    loss_ref[:, :] += jnp.full((1, 1), part)


def _et_body(e_ref, et_ref):
    e = e_ref[:]
    et_ref[:, :] = jnp.concatenate(
        [e.T, jnp.zeros((_K, 128 - _D), jnp.float32)], axis=1
    )


def _tc_stage(xf, embeddings):
    idx, loss = pl.pallas_call(
        _vq_tc_body,
        grid=(_N // _T,),
        in_specs=[
            pl.BlockSpec((_T, _D), lambda i: (i, 0)),
            pl.BlockSpec((_D, _K), lambda i: (0, 0)),
        ],
        out_specs=[
            pl.BlockSpec((1, 1, _T), lambda i: (i, 0, 0)),
            pl.BlockSpec((1, 1), lambda i: (0, 0)),
        ],
        out_shape=[
            jax.ShapeDtypeStruct((_N // _T, 1, _T), jnp.int32),
            jax.ShapeDtypeStruct((1, 1), jnp.float32),
        ],
    )(xf, embeddings)
    et = pl.pallas_call(
        _et_body,
        out_shape=jax.ShapeDtypeStruct((_K, 128), jnp.float32),
    )(embeddings)
    return idx, loss, et


_sc_mesh = plsc.VectorSubcoreMesh(core_axis_name="c", subcore_axis_name="s")


@functools.partial(
    pl.kernel,
    out_type=jax.ShapeDtypeStruct((_N, _D), jnp.float32),
    mesh=_sc_mesh,
    scratch_types=[
        pltpu.VMEM((_NCH, _CHUNK), jnp.int32),
        pltpu.VMEM((2, _CHUNK, 128), jnp.float32),
        pltpu.VMEM((_BW, _D), jnp.float32),
        pltpu.SemaphoreType.DMA,
        pltpu.SemaphoreType.DMA,
    ],
)
def _sc_gather(et_hbm, idx_hbm, out_hbm, idx_v, rows_v, out_v, sem0, sem1):
    wid = lax.axis_index("s") * _NC + lax.axis_index("c")
    base = wid * _BW
    sems = (sem0, sem1)
    pltpu.sync_copy(idx_hbm.at[pl.ds(wid * _NCH, _NCH)], idx_v)
    copies = [None, None]
    copies[0] = pltpu.async_copy(
        et_hbm.at[idx_v.at[0]], rows_v.at[0], sems[0]
    )
    for j in range(_NCH):
        b = j % 2
        copies[b].wait()
        if j + 1 < _NCH:
            nb = (j + 1) % 2
            copies[nb] = pltpu.async_copy(
                et_hbm.at[idx_v.at[j + 1]], rows_v.at[nb], sems[nb]
            )

        def _compact(t, _, j=j, b=b):
            for k in range(_D // 16):
                out_v[j * _CHUNK + t, pl.ds(k * 16, 16)] = rows_v[
                    b, t, pl.ds(k * 16, 16)
                ]
            return 0

        lax.fori_loop(0, _CHUNK, _compact, 0)
    pltpu.sync_copy(out_v, out_hbm.at[pl.ds(base, _BW)])


def kernel(x, embeddings):
    xf = x.reshape(_N, _D)
    idx, loss_sum, et = _tc_stage(xf, embeddings)
    q = _sc_gather(et, idx.reshape(_N // _CHUNK, _CHUNK))
    vq_loss = loss_sum[0, 0] * ((1.0 + _BETA) / (_N * _D))
    return q.reshape(x.shape), vq_loss


# speedup vs baseline: 1.7412x; 1.3883x over previous
import functools

import jax
import jax.numpy as jnp
from jax import lax
from jax.experimental import pallas as pl
from jax.experimental.pallas import tpu as pltpu
from jax.experimental.pallas import tpu_sc as plsc

_BETA = 0.25
_K = 1024
_D = 64
_T = 2048
_N = 16384

_NC = 2
_NS = 16
_NW = _NC * _NS
_BW = _N // _NW
_CHUNK = 128
_NCH = _BW // _CHUNK


def _vq_tc_body(x_ref, e_ref, idx_ref, loss_ref, et_ref):
    e = e_ref[:]
    xb = x_ref[:]
    sim2 = jnp.dot(xb * -2.0, e, preferred_element_type=jnp.float32)
    x2 = jnp.sum(xb * xb, axis=1, keepdims=True)
    e2 = jnp.sum(e * e, axis=0, keepdims=True)
    dist = (x2 + e2) + sim2
    iota = jax.lax.broadcasted_iota(jnp.int32, (_T, _K), 1).astype(jnp.float32)
    m = jnp.min(dist, axis=1, keepdims=True)
    idx_f = jnp.min(jnp.where(dist == m, iota, float(_K)), axis=1)
    idx_ref[:, :] = idx_f.astype(jnp.int32).reshape(_T // _CHUNK, _CHUNK)

    part = jnp.sum(m)

    @pl.when(pl.program_id(0) == 0)
    def _():
        loss_ref[:, :] = jnp.zeros((1, 1), jnp.float32)
        et_ref[:, :] = jnp.concatenate(
            [e.T, jnp.zeros((_K, 128 - _D), jnp.float32)], axis=1
        )

    loss_ref[:, :] += jnp.full((1, 1), part)


def _tc_stage(xf, embeddings):
    return pl.pallas_call(
        _vq_tc_body,
        grid=(_N // _T,),
        in_specs=[
            pl.BlockSpec((_T, _D), lambda i: (i, 0)),
            pl.BlockSpec((_D, _K), lambda i: (0, 0)),
        ],
        out_specs=[
            pl.BlockSpec((_T // _CHUNK, _CHUNK), lambda i: (i, 0)),
            pl.BlockSpec((1, 1), lambda i: (0, 0)),
            pl.BlockSpec((_K, 128), lambda i: (0, 0)),
        ],
        out_shape=[
            jax.ShapeDtypeStruct((_N // _CHUNK, _CHUNK), jnp.int32),
            jax.ShapeDtypeStruct((1, 1), jnp.float32),
            jax.ShapeDtypeStruct((_K, 128), jnp.float32),
        ],
    )(xf, embeddings)


_sc_mesh = plsc.VectorSubcoreMesh(core_axis_name="c", subcore_axis_name="s")


@functools.partial(
    pl.kernel,
    out_type=jax.ShapeDtypeStruct((_N, _D), jnp.float32),
    mesh=_sc_mesh,
    scratch_types=[
        pltpu.VMEM((_NCH, _CHUNK), jnp.int32),
        pltpu.VMEM((2, _CHUNK, 128), jnp.float32),
        pltpu.VMEM((_BW, _D), jnp.float32),
        pltpu.SemaphoreType.DMA,
        pltpu.SemaphoreType.DMA,
    ],
)
def _sc_gather(et_hbm, idx_hbm, out_hbm, idx_v, rows_v, out_v, sem0, sem1):
    wid = lax.axis_index("s") * _NC + lax.axis_index("c")
    base = wid * _BW
    sems = (sem0, sem1)
    pltpu.sync_copy(idx_hbm.at[pl.ds(wid * _NCH, _NCH)], idx_v)
    copies = [None, None]
    copies[0] = pltpu.async_copy(
        et_hbm.at[idx_v.at[0]], rows_v.at[0], sems[0]
    )
    for j in range(_NCH):
        b = j % 2
        copies[b].wait()
        if j + 1 < _NCH:
            nb = (j + 1) % 2
            copies[nb] = pltpu.async_copy(
                et_hbm.at[idx_v.at[j + 1]], rows_v.at[nb], sems[nb]
            )
        def _compact(t, _, j=j, b=b):
            for k in range(_D // 16):
                out_v[j * _CHUNK + t, pl.ds(k * 16, 16)] = rows_v[
                    b, t, pl.ds(k * 16, 16)
                ]
            return 0

        lax.fori_loop(0, _CHUNK, _compact, 0)
    pltpu.sync_copy(out_v, out_hbm.at[pl.ds(base, _BW)])


def kernel(x, embeddings):
    xf = x.reshape(_N, _D)
    idx, loss_sum, et = _tc_stage(xf, embeddings)
    q = _sc_gather(et, idx)
    vq_loss = loss_sum[0, 0] * ((1.0 + _BETA) / (_N * _D))
    return q.reshape(x.shape), vq_loss


# speedup vs baseline: 1.7683x; 1.0156x over previous
import functools

import jax
import jax.numpy as jnp
from jax import lax
from jax.experimental import pallas as pl
from jax.experimental.pallas import tpu as pltpu
from jax.experimental.pallas import tpu_sc as plsc

_BETA = 0.25
_K = 1024
_D = 64
_T = 4096
_N = 16384

_NC = 2
_NS = 16
_NW = _NC * _NS
_BW = _N // _NW
_CHUNK = 128
_NCH = _BW // _CHUNK


def _vq_tc_body(x_ref, e_ref, idx_ref, loss_ref, et_ref):
    e = e_ref[:]
    xb = x_ref[:]
    sim2 = jnp.dot(xb * -2.0, e, preferred_element_type=jnp.float32)
    x2 = jnp.sum(xb * xb, axis=1, keepdims=True)
    e2 = jnp.sum(e * e, axis=0, keepdims=True)
    dist = (x2 + e2) + sim2
    iota = jax.lax.broadcasted_iota(jnp.int32, (_T, _K), 1).astype(jnp.float32)
    m = jnp.min(dist, axis=1, keepdims=True)
    idx_f = jnp.min(jnp.where(dist == m, iota, float(_K)), axis=1)
    idx_ref[:, :] = idx_f.astype(jnp.int32).reshape(_T // _CHUNK, _CHUNK)

    part = jnp.sum(m)

    @pl.when(pl.program_id(0) == 0)
    def _():
        loss_ref[:, :] = jnp.zeros((1, 1), jnp.float32)
        et_ref[:, :] = jnp.concatenate(
            [e.T, jnp.zeros((_K, 128 - _D), jnp.float32)], axis=1
        )

    loss_ref[:, :] += jnp.full((1, 1), part)


def _tc_stage(xf, embeddings):
    return pl.pallas_call(
        _vq_tc_body,
        grid=(_N // _T,),
        in_specs=[
            pl.BlockSpec((_T, _D), lambda i: (i, 0)),
            pl.BlockSpec((_D, _K), lambda i: (0, 0)),
        ],
        out_specs=[
            pl.BlockSpec((_T // _CHUNK, _CHUNK), lambda i: (i, 0)),
            pl.BlockSpec((1, 1), lambda i: (0, 0)),
            pl.BlockSpec((_K, 128), lambda i: (0, 0)),
        ],
        out_shape=[
            jax.ShapeDtypeStruct((_N // _CHUNK, _CHUNK), jnp.int32),
            jax.ShapeDtypeStruct((1, 1), jnp.float32),
            jax.ShapeDtypeStruct((_K, 128), jnp.float32),
        ],
    )(xf, embeddings)


_sc_mesh = plsc.VectorSubcoreMesh(core_axis_name="c", subcore_axis_name="s")


@functools.partial(
    pl.kernel,
    out_type=jax.ShapeDtypeStruct((_N, _D), jnp.float32),
    mesh=_sc_mesh,
    scratch_types=[
        pltpu.VMEM((_NCH, _CHUNK), jnp.int32),
        pltpu.VMEM((2, _CHUNK, 128), jnp.float32),
        pltpu.VMEM((_BW, _D), jnp.float32),
        pltpu.SemaphoreType.DMA,
        pltpu.SemaphoreType.DMA,
    ],
)
def _sc_gather(et_hbm, idx_hbm, out_hbm, idx_v, rows_v, out_v, sem0, sem1):
    wid = lax.axis_index("s") * _NC + lax.axis_index("c")
    base = wid * _BW
    sems = (sem0, sem1)
    pltpu.sync_copy(idx_hbm.at[pl.ds(wid * _NCH, _NCH)], idx_v)
    copies = [None, None]
    copies[0] = pltpu.async_copy(
        et_hbm.at[idx_v.at[0]], rows_v.at[0], sems[0]
    )
    for j in range(_NCH):
        b = j % 2
        copies[b].wait()
        if j + 1 < _NCH:
            nb = (j + 1) % 2
            copies[nb] = pltpu.async_copy(
                et_hbm.at[idx_v.at[j + 1]], rows_v.at[nb], sems[nb]
            )
        def _compact(t, _, j=j, b=b):
            for k in range(_D // 16):
                out_v[j * _CHUNK + t, pl.ds(k * 16, 16)] = rows_v[
                    b, t, pl.ds(k * 16, 16)
                ]
            return 0

        lax.fori_loop(0, _CHUNK, _compact, 0)
    pltpu.sync_copy(out_v, out_hbm.at[pl.ds(base, _BW)])


def kernel(x, embeddings):
    xf = x.reshape(_N, _D)
    idx, loss_sum, et = _tc_stage(xf, embeddings)
    q = _sc_gather(et, idx)
    vq_loss = loss_sum[0, 0] * ((1.0 + _BETA) / (_N * _D))
    return q.reshape(x.shape), vq_loss
